# Initial kernel scaffold; baseline (speedup 1.0000x reference)
import jax
import jax.numpy as jnp
from jax.experimental import pallas as pl


def kernel(x, tables):
    raise NotImplementedError("write your pallas kernel here")


# speedup vs baseline: 26.4758x; 26.4758x over previous
import dataclasses
import functools

import numpy as np
import jax
import jax.numpy as jnp
from jax import lax
from jax.experimental import pallas as pl
from jax.experimental.pallas import tpu as pltpu
from jax.experimental.pallas import tpu_sc as plsc

_NUM_LEVELS = 16
_BASE_RES = 16
_MAX_RES = 2048
_HASHMAP_SIZE = 2 ** 19
_gb = np.exp((np.log(_MAX_RES) - np.log(_BASE_RES)) / (_NUM_LEVELS - 1))
_RESOLUTIONS = [int(np.floor(_BASE_RES * _gb ** i)) for i in range(_NUM_LEVELS)]
_TABLE_SIZES = [min(r * r, _HASHMAP_SIZE) for r in _RESOLUTIONS]
_P1 = 2654435761
_P2 = 805459861
_P1H, _P1L = _P1 >> 16, _P1 & 0xFFFF
_P2H, _P2L = _P2 >> 16, _P2 & 0xFFFF

_NC, _NS = 2, 16
_NW = _NC * _NS
_CHUNK = 128
_GRP = _CHUNK // 16
_F = 2
_OUTD = _NUM_LEVELS * _F


def _corner_pieces(c, ph, pl_):
    t = c * ph
    u = c * pl_
    lo = (t << 16) + u
    hi = (t + (u >> 16)) >> 16
    return hi, lo


def _combine(hi_a, lo_a, hi_b, lo_b, tsize):
    lo = lo_a ^ lo_b
    if tsize & (tsize - 1) == 0:
        return lo & (tsize - 1)
    hi = hi_a ^ hi_b
    m = lo & 0xFFFF
    nv = (lo >> 16) & 0xFFFF
    n1 = nv >> 8
    n0 = nv & 0xFF
    r32 = (1 << 32) % tsize
    r24 = (1 << 24) % tsize
    r16 = (1 << 16) % tsize
    s = hi * r32 + n1 * r24 + n0 * r16 + m
    inv_t = float(np.float32(1.0) / np.float32(tsize))
    q = (s.astype(jnp.float32) * inv_t).astype(jnp.int32)
    r = s - q * tsize
    r = jnp.where(r < 0, r + tsize, r)
    r = jnp.where(r >= tsize, r - tsize, r)
    return r


def _body(x_hbm, *refs):
    tables = refs[:_NUM_LEVELS]
    out_hbm = refs[_NUM_LEVELS]
    xb, normb, idxb, wb, rows, ob, sem = refs[_NUM_LEVELS + 1:]

    batch = x_hbm.shape[0] // 2
    rows_per_w = batch // _NW
    nchunks = rows_per_w // _CHUNK

    wid = (lax.axis_index("s").astype(jnp.int32) * jnp.int32(_NC)
           + lax.axis_index("c").astype(jnp.int32))
    iota = lax.iota(jnp.int32, 16)
    iota2 = iota * 2

    @pl.loop(0, nchunks)
    def _chunk(ci):
        base = wid * jnp.int32(rows_per_w) + ci.astype(jnp.int32) * jnp.int32(_CHUNK)
        pltpu.sync_copy(x_hbm.at[pl.ds(base * 2, 2 * _CHUNK)], xb)

        @pl.loop(0, _GRP)
        def _norm(g):
            g16 = g.astype(jnp.int32) * jnp.int32(16)
            g32 = g.astype(jnp.int32) * jnp.int32(32)
            xs = plsc.load_gather(xb, [iota2 + g32])
            ys = plsc.load_gather(xb, [iota2 + (g32 + jnp.int32(1))])
            nx = jnp.minimum(jnp.maximum(xs * 0.5 + 0.5, 0.0), 1.0)
            ny = jnp.minimum(jnp.maximum(ys * 0.5 + 0.5, 0.0), 1.0)
            normb[0, pl.ds(g16, 16)] = nx
            normb[1, pl.ds(g16, 16)] = ny

        copies = []
        for l in range(_NUM_LEVELS):
            res = _RESOLUTIONS[l]
            tsize = _TABLE_SIZES[l]

            @pl.loop(0, _GRP)
            def _idx(g, l=l, res=res, tsize=tsize):
                g16 = g.astype(jnp.int32) * jnp.int32(16)
                nx = normb[0, pl.ds(g16, 16)]
                ny = normb[1, pl.ds(g16, 16)]
                sx = nx * float(res)
                sy = ny * float(res)
                x0 = sx.astype(jnp.int32)
                y0 = sy.astype(jnp.int32)
                wx = sx - x0.astype(jnp.float32)
                wy = sy - y0.astype(jnp.float32)
                x1 = jnp.minimum(x0 + 1, res - 1)
                y1 = jnp.minimum(y0 + 1, res - 1)
                x0c = jnp.minimum(x0, res - 1)
                y0c = jnp.minimum(y0, res - 1)
                ha0, la0 = _corner_pieces(x0c, _P1H, _P1L)
                ha1, la1 = _corner_pieces(x1, _P1H, _P1L)
                hb0, lb0 = _corner_pieces(y0c, _P2H, _P2L)
                hb1, lb1 = _corner_pieces(y1, _P2H, _P2L)
                c00 = _combine(ha0, la0, hb0, lb0, tsize) * 2
                c10 = _combine(ha1, la1, hb0, lb0, tsize) * 2
                c01 = _combine(ha0, la0, hb1, lb1, tsize) * 2
                c11 = _combine(ha1, la1, hb1, lb1, tsize) * 2
                idxb[l, 0, pl.ds(g16, 16)] = c00
                idxb[l, 1, pl.ds(g16, 16)] = c00 + 1
                idxb[l, 2, pl.ds(g16, 16)] = c10
                idxb[l, 3, pl.ds(g16, 16)] = c10 + 1
                idxb[l, 4, pl.ds(g16, 16)] = c01
                idxb[l, 5, pl.ds(g16, 16)] = c01 + 1
                idxb[l, 6, pl.ds(g16, 16)] = c11
                idxb[l, 7, pl.ds(g16, 16)] = c11 + 1
                wb[l, 0, pl.ds(g16, 16)] = wx
                wb[l, 1, pl.ds(g16, 16)] = wy

            for jf in range(4 * _F):
                copies.append(pltpu.async_copy(
                    tables[l].at[idxb.at[l, jf]], rows.at[l, jf], sem))

        for l in range(_NUM_LEVELS):
            for jf in range(4 * _F):
                copies[l * 4 * _F + jf].wait()

            @pl.loop(0, _GRP)
            def _lerp(g, l=l):
                g16 = g.astype(jnp.int32) * jnp.int32(16)
                g512 = g.astype(jnp.int32) * jnp.int32(512)
                wx = wb[l, 0, pl.ds(g16, 16)]
                wy = wb[l, 1, pl.ds(g16, 16)]
                iota32 = iota * 32
                for f in range(_F):
                    e00 = rows[l, 0 + f, pl.ds(g16, 16)]
                    e10 = rows[l, 2 + f, pl.ds(g16, 16)]
                    e01 = rows[l, 4 + f, pl.ds(g16, 16)]
                    e11 = rows[l, 6 + f, pl.ds(g16, 16)]
                    va = e00 + (e10 - e00) * wx
                    vb = e01 + (e11 - e01) * wx
                    v = va + (vb - va) * wy
                    plsc.store_scatter(
                        ob, [iota32 + (g512 + jnp.int32(2 * l + f))], v)

        pltpu.sync_copy(ob, out_hbm.at[pl.ds(base * 32, 32 * _CHUNK)])


def kernel(x, tables):
    batch = x.shape[0]
    assert batch % (_NW * _CHUNK) == 0
    from jax._src import config as _jcfg
    with _jcfg.enable_x64(False):
        return _build_and_run(x, tables, batch)


def _build_and_run(x, tables, batch):
    mesh = plsc.VectorSubcoreMesh(core_axis_name="c", subcore_axis_name="s")
    cp = pltpu.CompilerParams()
    if "needs_layout_passes" in pltpu.CompilerParams.__dataclass_fields__:
        cp = dataclasses.replace(cp, needs_layout_passes=False)
    if "use_tc_tiling_on_sc" in pltpu.CompilerParams.__dataclass_fields__:
        cp = dataclasses.replace(cp, use_tc_tiling_on_sc=False)
    run = pl.kernel(
        _body,
        out_type=jax.ShapeDtypeStruct((batch * _OUTD,), jnp.float32),
        mesh=mesh,
        scratch_types=[
            pltpu.VMEM((2 * _CHUNK,), jnp.float32),
            pltpu.VMEM((2, _CHUNK), jnp.float32),
            pltpu.VMEM((_NUM_LEVELS, 4 * _F, _CHUNK), jnp.int32),
            pltpu.VMEM((_NUM_LEVELS, 2, _CHUNK), jnp.float32),
            pltpu.VMEM((_NUM_LEVELS, 4 * _F, _CHUNK), jnp.float32),
            pltpu.VMEM((_OUTD * _CHUNK,), jnp.float32),
            pltpu.SemaphoreType.DMA,
        ],
        compiler_params=cp,
    )
    x_flat = x.reshape(-1)
    tbl_flat = tuple(t.reshape(-1) for t in tables)
    out_flat = run(x_flat, *tbl_flat)
    return out_flat.reshape(batch, _OUTD)


# speedup vs baseline: 55.1273x; 2.0822x over previous
import dataclasses
import functools

import numpy as np
import jax
import jax.numpy as jnp
from jax import lax
from jax.experimental import pallas as pl
from jax.experimental.pallas import tpu as pltpu
from jax.experimental.pallas import tpu_sc as plsc

_NUM_LEVELS = 16
_BASE_RES = 16
_MAX_RES = 2048
_HASHMAP_SIZE = 2 ** 19
_gb = np.exp((np.log(_MAX_RES) - np.log(_BASE_RES)) / (_NUM_LEVELS - 1))
_RESOLUTIONS = [int(np.floor(_BASE_RES * _gb ** i)) for i in range(_NUM_LEVELS)]
_TABLE_SIZES = [min(r * r, _HASHMAP_SIZE) for r in _RESOLUTIONS]
_P1 = 2654435761
_P2 = 805459861
_P1H, _P1L = _P1 >> 16, _P1 & 0xFFFF
_P2H, _P2L = _P2 >> 16, _P2 & 0xFFFF

_NC, _NS = 2, 16
_NW = _NC * _NS
_CHUNK = 128
_GRP = _CHUNK // 16
_F = 2
_OUTD = _NUM_LEVELS * _F
_NSMALL = 7
_NBIG = _NUM_LEVELS - _NSMALL
_TBL_OFF = []
_acc = 0
for _t in _TABLE_SIZES[:_NSMALL]:
    _TBL_OFF.append(_acc)
    _acc += 2 * _t
_TBLV_WORDS = _acc


def _corner_pieces(c, ph, pl_):
    t = c * ph
    u = c * pl_
    lo = (t << 16) + u
    hi = (t + (u >> 16)) >> 16
    return hi, lo


def _combine(hi_a, lo_a, hi_b, lo_b, tsize):
    lo = lo_a ^ lo_b
    if tsize & (tsize - 1) == 0:
        return lo & (tsize - 1)
    hi = hi_a ^ hi_b
    m = lo & 0xFFFF
    nv = (lo >> 16) & 0xFFFF
    n1 = nv >> 8
    n0 = nv & 0xFF
    r32 = (1 << 32) % tsize
    r24 = (1 << 24) % tsize
    r16 = (1 << 16) % tsize
    s = hi * r32 + n1 * r24 + n0 * r16 + m
    inv_t = float(np.float32(1.0) / np.float32(tsize))
    q = (s.astype(jnp.float32) * inv_t).astype(jnp.int32)
    r = s - q * tsize
    r = jnp.where(r < 0, r + tsize, r)
    r = jnp.where(r >= tsize, r - tsize, r)
    return r


def _body(x_hbm, *refs):
    tables = refs[:_NUM_LEVELS]
    out_hbm = refs[_NUM_LEVELS]
    xb, normb, tblv, idxb, wb, rows, ob, sem = refs[_NUM_LEVELS + 1:]

    batch = x_hbm.shape[0] // 2
    rows_per_w = batch // _NW
    nchunks = rows_per_w // _CHUNK

    wid = (lax.axis_index("s").astype(jnp.int32) * jnp.int32(_NC)
           + lax.axis_index("c").astype(jnp.int32))
    iota = lax.iota(jnp.int32, 16)
    iota2 = iota * 2

    for l in range(_NSMALL):
        pltpu.sync_copy(tables[l], tblv.at[pl.ds(_TBL_OFF[l], 2 * _TABLE_SIZES[l])])

    @pl.loop(0, nchunks)
    def _chunk(ci):
        base = wid * jnp.int32(rows_per_w) + ci.astype(jnp.int32) * jnp.int32(_CHUNK)
        pltpu.sync_copy(x_hbm.at[pl.ds(base * 2, 2 * _CHUNK)], xb)

        @pl.loop(0, _GRP)
        def _norm(g):
            g16 = g.astype(jnp.int32) * jnp.int32(16)
            g32 = g.astype(jnp.int32) * jnp.int32(32)
            xs = plsc.load_gather(xb, [iota2 + g32])
            ys = plsc.load_gather(xb, [iota2 + (g32 + jnp.int32(1))])
            nx = jnp.minimum(jnp.maximum(xs * 0.5 + 0.5, 0.0), 1.0)
            ny = jnp.minimum(jnp.maximum(ys * 0.5 + 0.5, 0.0), 1.0)
            normb[0, pl.ds(g16, 16)] = nx
            normb[1, pl.ds(g16, 16)] = ny

        for l in range(_NSMALL):
            res = _RESOLUTIONS[l]
            tsize = _TABLE_SIZES[l]
            off = _TBL_OFF[l]

            @pl.loop(0, _GRP)
            def _small(g, l=l, res=res, tsize=tsize, off=off):
                g16 = g.astype(jnp.int32) * jnp.int32(16)
                g512 = g.astype(jnp.int32) * jnp.int32(512)
                nx = normb[0, pl.ds(g16, 16)]
                ny = normb[1, pl.ds(g16, 16)]
                sx = nx * float(res)
                sy = ny * float(res)
                x0 = sx.astype(jnp.int32)
                y0 = sy.astype(jnp.int32)
                wx = sx - x0.astype(jnp.float32)
                wy = sy - y0.astype(jnp.float32)
                x1 = jnp.minimum(x0 + 1, res - 1)
                y1 = jnp.minimum(y0 + 1, res - 1)
                x0c = jnp.minimum(x0, res - 1)
                y0c = jnp.minimum(y0, res - 1)
                ha0, la0 = _corner_pieces(x0c, _P1H, _P1L)
                ha1, la1 = _corner_pieces(x1, _P1H, _P1L)
                hb0, lb0 = _corner_pieces(y0c, _P2H, _P2L)
                hb1, lb1 = _corner_pieces(y1, _P2H, _P2L)
                c00 = _combine(ha0, la0, hb0, lb0, tsize) * 2 + off
                c10 = _combine(ha1, la1, hb0, lb0, tsize) * 2 + off
                c01 = _combine(ha0, la0, hb1, lb1, tsize) * 2 + off
                c11 = _combine(ha1, la1, hb1, lb1, tsize) * 2 + off
                iota32 = iota * 32
                for f in range(_F):
                    e00 = plsc.load_gather(tblv, [c00 + f])
                    e10 = plsc.load_gather(tblv, [c10 + f])
                    e01 = plsc.load_gather(tblv, [c01 + f])
                    e11 = plsc.load_gather(tblv, [c11 + f])
                    va = e00 + (e10 - e00) * wx
                    vb = e01 + (e11 - e01) * wx
                    v = va + (vb - va) * wy
                    plsc.store_scatter(
                        ob, [iota32 + (g512 + jnp.int32(2 * l + f))], v)

        copies = []
        for l in range(_NSMALL, _NUM_LEVELS):
            res = _RESOLUTIONS[l]
            tsize = _TABLE_SIZES[l]
            bl = l - _NSMALL

            @pl.loop(0, _GRP)
            def _idx(g, bl=bl, res=res, tsize=tsize):
                g16 = g.astype(jnp.int32) * jnp.int32(16)
                nx = normb[0, pl.ds(g16, 16)]
                ny = normb[1, pl.ds(g16, 16)]
                sx = nx * float(res)
                sy = ny * float(res)
                x0 = sx.astype(jnp.int32)
                y0 = sy.astype(jnp.int32)
                wx = sx - x0.astype(jnp.float32)
                wy = sy - y0.astype(jnp.float32)
                x1 = jnp.minimum(x0 + 1, res - 1)
                y1 = jnp.minimum(y0 + 1, res - 1)
                x0c = jnp.minimum(x0, res - 1)
                y0c = jnp.minimum(y0, res - 1)
                ha0, la0 = _corner_pieces(x0c, _P1H, _P1L)
                ha1, la1 = _corner_pieces(x1, _P1H, _P1L)
                hb0, lb0 = _corner_pieces(y0c, _P2H, _P2L)
                hb1, lb1 = _corner_pieces(y1, _P2H, _P2L)
                c00 = _combine(ha0, la0, hb0, lb0, tsize) * 2
                c10 = _combine(ha1, la1, hb0, lb0, tsize) * 2
                c01 = _combine(ha0, la0, hb1, lb1, tsize) * 2
                c11 = _combine(ha1, la1, hb1, lb1, tsize) * 2
                idxb[bl, 0, pl.ds(g16, 16)] = c00
                idxb[bl, 1, pl.ds(g16, 16)] = c00 + 1
                idxb[bl, 2, pl.ds(g16, 16)] = c10
                idxb[bl, 3, pl.ds(g16, 16)] = c10 + 1
                idxb[bl, 4, pl.ds(g16, 16)] = c01
                idxb[bl, 5, pl.ds(g16, 16)] = c01 + 1
                idxb[bl, 6, pl.ds(g16, 16)] = c11
                idxb[bl, 7, pl.ds(g16, 16)] = c11 + 1
                wb[bl, 0, pl.ds(g16, 16)] = wx
                wb[bl, 1, pl.ds(g16, 16)] = wy

            for jf in range(4 * _F):
                copies.append(pltpu.async_copy(
                    tables[l].at[idxb.at[bl, jf]], rows.at[bl, jf], sem))

        for l in range(_NSMALL, _NUM_LEVELS):
            bl = l - _NSMALL
            for jf in range(4 * _F):
                copies[bl * 4 * _F + jf].wait()

            @pl.loop(0, _GRP)
            def _lerp(g, l=l, bl=bl):
                g16 = g.astype(jnp.int32) * jnp.int32(16)
                g512 = g.astype(jnp.int32) * jnp.int32(512)
                wx = wb[bl, 0, pl.ds(g16, 16)]
                wy = wb[bl, 1, pl.ds(g16, 16)]
                iota32 = iota * 32
                for f in range(_F):
                    e00 = rows[bl, 0 + f, pl.ds(g16, 16)]
                    e10 = rows[bl, 2 + f, pl.ds(g16, 16)]
                    e01 = rows[bl, 4 + f, pl.ds(g16, 16)]
                    e11 = rows[bl, 6 + f, pl.ds(g16, 16)]
                    va = e00 + (e10 - e00) * wx
                    vb = e01 + (e11 - e01) * wx
                    v = va + (vb - va) * wy
                    plsc.store_scatter(
                        ob, [iota32 + (g512 + jnp.int32(2 * l + f))], v)

        pltpu.sync_copy(ob, out_hbm.at[pl.ds(base * 32, 32 * _CHUNK)])


def kernel(x, tables):
    batch = x.shape[0]
    assert batch % (_NW * _CHUNK) == 0
    from jax._src import config as _jcfg
    with _jcfg.enable_x64(False):
        return _build_and_run(x, tables, batch)


def _build_and_run(x, tables, batch):
    mesh = plsc.VectorSubcoreMesh(core_axis_name="c", subcore_axis_name="s")
    cp = pltpu.CompilerParams()
    if "needs_layout_passes" in pltpu.CompilerParams.__dataclass_fields__:
        cp = dataclasses.replace(cp, needs_layout_passes=False)
    if "use_tc_tiling_on_sc" in pltpu.CompilerParams.__dataclass_fields__:
        cp = dataclasses.replace(cp, use_tc_tiling_on_sc=False)
    run = pl.kernel(
        _body,
        out_type=jax.ShapeDtypeStruct((batch * _OUTD,), jnp.float32),
        mesh=mesh,
        scratch_types=[
            pltpu.VMEM((2 * _CHUNK,), jnp.float32),
            pltpu.VMEM((2, _CHUNK), jnp.float32),
            pltpu.VMEM((_TBLV_WORDS,), jnp.float32),
            pltpu.VMEM((_NBIG, 4 * _F, _CHUNK), jnp.int32),
            pltpu.VMEM((_NBIG, 2, _CHUNK), jnp.float32),
            pltpu.VMEM((_NBIG, 4 * _F, _CHUNK), jnp.float32),
            pltpu.VMEM((_OUTD * _CHUNK,), jnp.float32),
            pltpu.SemaphoreType.DMA,
        ],
        compiler_params=cp,
    )
    x_flat = x.reshape(-1)
    tbl_flat = tuple(t.reshape(-1) for t in tables)
    out_flat = run(x_flat, *tbl_flat)
    return out_flat.reshape(batch, _OUTD)


# speedup vs baseline: 55.3496x; 1.0040x over previous
import dataclasses
import functools

import numpy as np
import jax
import jax.numpy as jnp
from jax import lax
from jax.experimental import pallas as pl
from jax.experimental.pallas import tpu as pltpu
from jax.experimental.pallas import tpu_sc as plsc

_NUM_LEVELS = 16
_BASE_RES = 16
_MAX_RES = 2048
_HASHMAP_SIZE = 2 ** 19
_gb = np.exp((np.log(_MAX_RES) - np.log(_BASE_RES)) / (_NUM_LEVELS - 1))
_RESOLUTIONS = [int(np.floor(_BASE_RES * _gb ** i)) for i in range(_NUM_LEVELS)]
_TABLE_SIZES = [min(r * r, _HASHMAP_SIZE) for r in _RESOLUTIONS]
_P1 = 2654435761
_P2 = 805459861
_P1H, _P1L = _P1 >> 16, _P1 & 0xFFFF
_P2H, _P2L = _P2 >> 16, _P2 & 0xFFFF

_NC, _NS = 2, 16
_NW = _NC * _NS
_CHUNK = 128
_GRP = _CHUNK // 16
_F = 2
_OUTD = _NUM_LEVELS * _F
_NSMALL = 7
_NBIG = _NUM_LEVELS - _NSMALL
_TBL_OFF = []
_acc = 0
for _t in _TABLE_SIZES[:_NSMALL]:
    _TBL_OFF.append(_acc)
    _acc += 2 * _t
_TBLV_WORDS = _acc


def _corner_pieces(c, ph, pl_):
    t = c * ph
    u = c * pl_
    lo = (t << 16) + u
    hi = (t + (u >> 16)) >> 16
    return hi, lo


def _combine(hi_a, lo_a, hi_b, lo_b, tsize):
    lo = lo_a ^ lo_b
    if tsize & (tsize - 1) == 0:
        return lo & (tsize - 1)
    hi = hi_a ^ hi_b
    m = lo & 0xFFFF
    nv = (lo >> 16) & 0xFFFF
    n1 = nv >> 8
    n0 = nv & 0xFF
    r32 = (1 << 32) % tsize
    r24 = (1 << 24) % tsize
    r16 = (1 << 16) % tsize
    s = hi * r32 + n1 * r24 + n0 * r16 + m
    inv_t = float(np.float32(1.0) / np.float32(tsize))
    q = (s.astype(jnp.float32) * inv_t).astype(jnp.int32)
    r = s - q * tsize
    r = jnp.where(r < 0, r + tsize, r)
    r = jnp.where(r >= tsize, r - tsize, r)
    return r


def _body(x_hbm, *refs):
    tables = refs[:_NUM_LEVELS]
    out_hbm = refs[_NUM_LEVELS]
    xb, normb, tblv, idxb, wb, rows, ob, sem = refs[_NUM_LEVELS + 1:]

    batch = x_hbm.shape[0] // 2
    rows_per_w = batch // _NW
    nchunks = rows_per_w // _CHUNK

    wid = (lax.axis_index("s").astype(jnp.int32) * jnp.int32(_NC)
           + lax.axis_index("c").astype(jnp.int32))
    iota = lax.iota(jnp.int32, 16)
    iota2 = iota * 2

    for l in range(_NSMALL):
        pltpu.sync_copy(tables[l], tblv.at[pl.ds(_TBL_OFF[l], 2 * _TABLE_SIZES[l])])

    @pl.loop(0, nchunks)
    def _chunk(ci):
        base = wid * jnp.int32(rows_per_w) + ci.astype(jnp.int32) * jnp.int32(_CHUNK)
        pltpu.sync_copy(x_hbm.at[pl.ds(base * 2, 2 * _CHUNK)], xb)

        @pl.loop(0, _GRP)
        def _norm(g):
            g16 = g.astype(jnp.int32) * jnp.int32(16)
            g32 = g.astype(jnp.int32) * jnp.int32(32)
            xs = plsc.load_gather(xb, [iota2 + g32])
            ys = plsc.load_gather(xb, [iota2 + (g32 + jnp.int32(1))])
            nx = jnp.minimum(jnp.maximum(xs * 0.5 + 0.5, 0.0), 1.0)
            ny = jnp.minimum(jnp.maximum(ys * 0.5 + 0.5, 0.0), 1.0)
            normb[0, pl.ds(g16, 16)] = nx
            normb[1, pl.ds(g16, 16)] = ny

        for l in range(_NSMALL):
            res = _RESOLUTIONS[l]
            tsize = _TABLE_SIZES[l]
            off = _TBL_OFF[l]

            @pl.loop(0, _GRP)
            def _small(g, l=l, res=res, tsize=tsize, off=off):
                g16 = g.astype(jnp.int32) * jnp.int32(16)
                g512 = g.astype(jnp.int32) * jnp.int32(512)
                nx = normb[0, pl.ds(g16, 16)]
                ny = normb[1, pl.ds(g16, 16)]
                sx = nx * float(res)
                sy = ny * float(res)
                x0 = sx.astype(jnp.int32)
                y0 = sy.astype(jnp.int32)
                wx = sx - x0.astype(jnp.float32)
                wy = sy - y0.astype(jnp.float32)
                x1 = jnp.minimum(x0 + 1, res - 1)
                y1 = jnp.minimum(y0 + 1, res - 1)
                x0c = jnp.minimum(x0, res - 1)
                y0c = jnp.minimum(y0, res - 1)
                ha0, la0 = _corner_pieces(x0c, _P1H, _P1L)
                ha1, la1 = _corner_pieces(x1, _P1H, _P1L)
                hb0, lb0 = _corner_pieces(y0c, _P2H, _P2L)
                hb1, lb1 = _corner_pieces(y1, _P2H, _P2L)
                c00 = _combine(ha0, la0, hb0, lb0, tsize) * 2 + off
                c10 = _combine(ha1, la1, hb0, lb0, tsize) * 2 + off
                c01 = _combine(ha0, la0, hb1, lb1, tsize) * 2 + off
                c11 = _combine(ha1, la1, hb1, lb1, tsize) * 2 + off
                iota32 = iota * 32
                for f in range(_F):
                    e00 = plsc.load_gather(tblv, [c00 + f])
                    e10 = plsc.load_gather(tblv, [c10 + f])
                    e01 = plsc.load_gather(tblv, [c01 + f])
                    e11 = plsc.load_gather(tblv, [c11 + f])
                    va = e00 + (e10 - e00) * wx
                    vb = e01 + (e11 - e01) * wx
                    v = va + (vb - va) * wy
                    plsc.store_scatter(
                        ob, [iota32 + (g512 + jnp.int32(2 * l + f))], v)

        copies = []
        for l in range(_NSMALL, _NUM_LEVELS):
            res = _RESOLUTIONS[l]
            tsize = _TABLE_SIZES[l]
            bl = l - _NSMALL

            @pl.loop(0, _GRP)
            def _idx(g, bl=bl, res=res, tsize=tsize):
                g16 = g.astype(jnp.int32) * jnp.int32(16)
                nx = normb[0, pl.ds(g16, 16)]
                ny = normb[1, pl.ds(g16, 16)]
                sx = nx * float(res)
                sy = ny * float(res)
                x0 = sx.astype(jnp.int32)
                y0 = sy.astype(jnp.int32)
                wx = sx - x0.astype(jnp.float32)
                wy = sy - y0.astype(jnp.float32)
                x1 = jnp.minimum(x0 + 1, res - 1)
                y1 = jnp.minimum(y0 + 1, res - 1)
                x0c = jnp.minimum(x0, res - 1)
                y0c = jnp.minimum(y0, res - 1)
                ha0, la0 = _corner_pieces(x0c, _P1H, _P1L)
                ha1, la1 = _corner_pieces(x1, _P1H, _P1L)
                hb0, lb0 = _corner_pieces(y0c, _P2H, _P2L)
                hb1, lb1 = _corner_pieces(y1, _P2H, _P2L)
                c00 = _combine(ha0, la0, hb0, lb0, tsize) * 2
                c10 = _combine(ha1, la1, hb0, lb0, tsize) * 2
                c01 = _combine(ha0, la0, hb1, lb1, tsize) * 2
                c11 = _combine(ha1, la1, hb1, lb1, tsize) * 2
                idxb[bl, pl.ds(g16, 16)] = c00
                idxb[bl, pl.ds(g16 + jnp.int32(128), 16)] = c00 + 1
                idxb[bl, pl.ds(g16 + jnp.int32(256), 16)] = c10
                idxb[bl, pl.ds(g16 + jnp.int32(384), 16)] = c10 + 1
                idxb[bl, pl.ds(g16 + jnp.int32(512), 16)] = c01
                idxb[bl, pl.ds(g16 + jnp.int32(640), 16)] = c01 + 1
                idxb[bl, pl.ds(g16 + jnp.int32(768), 16)] = c11
                idxb[bl, pl.ds(g16 + jnp.int32(896), 16)] = c11 + 1
                wb[bl, 0, pl.ds(g16, 16)] = wx
                wb[bl, 1, pl.ds(g16, 16)] = wy

            copies.append(pltpu.async_copy(
                tables[l].at[idxb.at[bl]], rows.at[bl], sem))

        for l in range(_NSMALL, _NUM_LEVELS):
            bl = l - _NSMALL
            copies[bl].wait()

            @pl.loop(0, _GRP)
            def _lerp(g, l=l, bl=bl):
                g16 = g.astype(jnp.int32) * jnp.int32(16)
                g512 = g.astype(jnp.int32) * jnp.int32(512)
                wx = wb[bl, 0, pl.ds(g16, 16)]
                wy = wb[bl, 1, pl.ds(g16, 16)]
                iota32 = iota * 32
                for f in range(_F):
                    e00 = rows[bl, pl.ds(g16 + jnp.int32((0 + f) * 128), 16)]
                    e10 = rows[bl, pl.ds(g16 + jnp.int32((2 + f) * 128), 16)]
                    e01 = rows[bl, pl.ds(g16 + jnp.int32((4 + f) * 128), 16)]
                    e11 = rows[bl, pl.ds(g16 + jnp.int32((6 + f) * 128), 16)]
                    va = e00 + (e10 - e00) * wx
                    vb = e01 + (e11 - e01) * wx
                    v = va + (vb - va) * wy
                    plsc.store_scatter(
                        ob, [iota32 + (g512 + jnp.int32(2 * l + f))], v)

        pltpu.sync_copy(ob, out_hbm.at[pl.ds(base * 32, 32 * _CHUNK)])


def kernel(x, tables):
    batch = x.shape[0]
    assert batch % (_NW * _CHUNK) == 0
    from jax._src import config as _jcfg
    with _jcfg.enable_x64(False):
        return _build_and_run(x, tables, batch)


def _build_and_run(x, tables, batch):
    mesh = plsc.VectorSubcoreMesh(core_axis_name="c", subcore_axis_name="s")
    cp = pltpu.CompilerParams()
    if "needs_layout_passes" in pltpu.CompilerParams.__dataclass_fields__:
        cp = dataclasses.replace(cp, needs_layout_passes=False)
    if "use_tc_tiling_on_sc" in pltpu.CompilerParams.__dataclass_fields__:
        cp = dataclasses.replace(cp, use_tc_tiling_on_sc=False)
    run = pl.kernel(
        _body,
        out_type=jax.ShapeDtypeStruct((batch * _OUTD,), jnp.float32),
        mesh=mesh,
        scratch_types=[
            pltpu.VMEM((2 * _CHUNK,), jnp.float32),
            pltpu.VMEM((2, _CHUNK), jnp.float32),
            pltpu.VMEM((_TBLV_WORDS,), jnp.float32),
            pltpu.VMEM((_NBIG, 4 * _F * _CHUNK), jnp.int32),
            pltpu.VMEM((_NBIG, 2, _CHUNK), jnp.float32),
            pltpu.VMEM((_NBIG, 4 * _F * _CHUNK), jnp.float32),
            pltpu.VMEM((_OUTD * _CHUNK,), jnp.float32),
            pltpu.SemaphoreType.DMA,
        ],
        compiler_params=cp,
    )
    x_flat = x.reshape(-1)
    tbl_flat = tuple(t.reshape(-1) for t in tables)
    out_flat = run(x_flat, *tbl_flat)
    return out_flat.reshape(batch, _OUTD)


# speedup vs baseline: 65.3218x; 1.1802x over previous
import dataclasses
import functools

import numpy as np
import jax
import jax.numpy as jnp
from jax import lax
from jax.experimental import pallas as pl
from jax.experimental.pallas import tpu as pltpu
from jax.experimental.pallas import tpu_sc as plsc

_NUM_LEVELS = 16
_BASE_RES = 16
_MAX_RES = 2048
_HASHMAP_SIZE = 2 ** 19
_gb = np.exp((np.log(_MAX_RES) - np.log(_BASE_RES)) / (_NUM_LEVELS - 1))
_RESOLUTIONS = [int(np.floor(_BASE_RES * _gb ** i)) for i in range(_NUM_LEVELS)]
_TABLE_SIZES = [min(r * r, _HASHMAP_SIZE) for r in _RESOLUTIONS]
_P1 = 2654435761
_P2 = 805459861
_P1H, _P1L = _P1 >> 16, _P1 & 0xFFFF
_P2H, _P2L = _P2 >> 16, _P2 & 0xFFFF

_NC, _NS = 2, 16
_NW = _NC * _NS
_CHUNK = 128
_GRP = _CHUNK // 16
_F = 2
_OUTD = _NUM_LEVELS * _F
_NSMALL = 7
_NBIG = _NUM_LEVELS - _NSMALL
_TBL_OFF = []
_acc = 0
for _t in _TABLE_SIZES[:_NSMALL]:
    _TBL_OFF.append(_acc)
    _acc += 2 * _t
_TBLV_WORDS = _acc


def _corner_pieces(c, ph, pl_):
    t = c * ph
    u = c * pl_
    lo = (t << 16) + u
    hi = (t + (u >> 16)) >> 16
    return hi, lo


def _combine(hi_a, lo_a, hi_b, lo_b, tsize):
    lo = lo_a ^ lo_b
    if tsize & (tsize - 1) == 0:
        return lo & (tsize - 1)
    hi = hi_a ^ hi_b
    m = lo & 0xFFFF
    nv = (lo >> 16) & 0xFFFF
    n1 = nv >> 8
    n0 = nv & 0xFF
    r32 = (1 << 32) % tsize
    r24 = (1 << 24) % tsize
    r16 = (1 << 16) % tsize
    s = hi * r32 + n1 * r24 + n0 * r16 + m
    inv_t = float(np.float32(1.0) / np.float32(tsize))
    q = (s.astype(jnp.float32) * inv_t).astype(jnp.int32)
    r = s - q * tsize
    r = jnp.where(r < 0, r + tsize, r)
    r = jnp.where(r >= tsize, r - tsize, r)
    return r


def _body(x_hbm, *refs):
    tables = refs[:_NUM_LEVELS]
    out_hbm = refs[_NUM_LEVELS]
    xb, normb, tblv, idxb, wb, rows, ob, sem = refs[_NUM_LEVELS + 1:]

    batch = x_hbm.shape[0] // 2
    rows_per_w = batch // _NW
    nchunks = rows_per_w // _CHUNK

    wid = (lax.axis_index("s").astype(jnp.int32) * jnp.int32(_NC)
           + lax.axis_index("c").astype(jnp.int32))
    iota = lax.iota(jnp.int32, 16)
    iota2 = iota * 2

    for l in range(_NSMALL):
        pltpu.sync_copy(tables[l], tblv.at[pl.ds(_TBL_OFF[l], 2 * _TABLE_SIZES[l])])

    @pl.loop(0, nchunks)
    def _chunk(ci):
        base = wid * jnp.int32(rows_per_w) + ci.astype(jnp.int32) * jnp.int32(_CHUNK)
        pltpu.sync_copy(x_hbm.at[pl.ds(base * 2, 2 * _CHUNK)], xb)

        @pl.loop(0, _GRP)
        def _norm(g):
            g16 = g.astype(jnp.int32) * jnp.int32(16)
            g32 = g.astype(jnp.int32) * jnp.int32(32)
            xs = plsc.load_gather(xb, [iota2 + g32])
            ys = plsc.load_gather(xb, [iota2 + (g32 + jnp.int32(1))])
            nx = jnp.minimum(jnp.maximum(xs * 0.5 + 0.5, 0.0), 1.0)
            ny = jnp.minimum(jnp.maximum(ys * 0.5 + 0.5, 0.0), 1.0)
            normb[0, pl.ds(g16, 16)] = nx
            normb[1, pl.ds(g16, 16)] = ny

        copies = []
        for l in range(_NSMALL, _NUM_LEVELS):
            res = _RESOLUTIONS[l]
            tsize = _TABLE_SIZES[l]
            bl = l - _NSMALL

            @pl.loop(0, _GRP)
            def _idx(g, bl=bl, res=res, tsize=tsize):
                g16 = g.astype(jnp.int32) * jnp.int32(16)
                nx = normb[0, pl.ds(g16, 16)]
                ny = normb[1, pl.ds(g16, 16)]
                sx = nx * float(res)
                sy = ny * float(res)
                x0 = sx.astype(jnp.int32)
                y0 = sy.astype(jnp.int32)
                wx = sx - x0.astype(jnp.float32)
                wy = sy - y0.astype(jnp.float32)
                x1 = jnp.minimum(x0 + 1, res - 1)
                y1 = jnp.minimum(y0 + 1, res - 1)
                x0c = jnp.minimum(x0, res - 1)
                y0c = jnp.minimum(y0, res - 1)
                ha0, la0 = _corner_pieces(x0c, _P1H, _P1L)
                ha1, la1 = _corner_pieces(x1, _P1H, _P1L)
                hb0, lb0 = _corner_pieces(y0c, _P2H, _P2L)
                hb1, lb1 = _corner_pieces(y1, _P2H, _P2L)
                c00 = _combine(ha0, la0, hb0, lb0, tsize) * 2
                c10 = _combine(ha1, la1, hb0, lb0, tsize) * 2
                c01 = _combine(ha0, la0, hb1, lb1, tsize) * 2
                c11 = _combine(ha1, la1, hb1, lb1, tsize) * 2
                idxb[bl, pl.ds(g16, 16)] = c00
                idxb[bl, pl.ds(g16 + jnp.int32(128), 16)] = c00 + 1
                idxb[bl, pl.ds(g16 + jnp.int32(256), 16)] = c10
                idxb[bl, pl.ds(g16 + jnp.int32(384), 16)] = c10 + 1
                idxb[bl, pl.ds(g16 + jnp.int32(512), 16)] = c01
                idxb[bl, pl.ds(g16 + jnp.int32(640), 16)] = c01 + 1
                idxb[bl, pl.ds(g16 + jnp.int32(768), 16)] = c11
                idxb[bl, pl.ds(g16 + jnp.int32(896), 16)] = c11 + 1
                wb[bl, 0, pl.ds(g16, 16)] = wx
                wb[bl, 1, pl.ds(g16, 16)] = wy

            copies.append(pltpu.async_copy(
                tables[l].at[idxb.at[bl]], rows.at[bl], sem))

        for l in range(_NSMALL):
            res = _RESOLUTIONS[l]
            tsize = _TABLE_SIZES[l]
            off = _TBL_OFF[l]

            @pl.loop(0, _GRP)
            def _small(g, l=l, res=res, tsize=tsize, off=off):
                g16 = g.astype(jnp.int32) * jnp.int32(16)
                g512 = g.astype(jnp.int32) * jnp.int32(512)
                nx = normb[0, pl.ds(g16, 16)]
                ny = normb[1, pl.ds(g16, 16)]
                sx = nx * float(res)
                sy = ny * float(res)
                x0 = sx.astype(jnp.int32)
                y0 = sy.astype(jnp.int32)
                wx = sx - x0.astype(jnp.float32)
                wy = sy - y0.astype(jnp.float32)
                x1 = jnp.minimum(x0 + 1, res - 1)
                y1 = jnp.minimum(y0 + 1, res - 1)
                x0c = jnp.minimum(x0, res - 1)
                y0c = jnp.minimum(y0, res - 1)
                ha0, la0 = _corner_pieces(x0c, _P1H, _P1L)
                ha1, la1 = _corner_pieces(x1, _P1H, _P1L)
                hb0, lb0 = _corner_pieces(y0c, _P2H, _P2L)
                hb1, lb1 = _corner_pieces(y1, _P2H, _P2L)
                c00 = _combine(ha0, la0, hb0, lb0, tsize) * 2 + off
                c10 = _combine(ha1, la1, hb0, lb0, tsize) * 2 + off
                c01 = _combine(ha0, la0, hb1, lb1, tsize) * 2 + off
                c11 = _combine(ha1, la1, hb1, lb1, tsize) * 2 + off
                iota32 = iota * 32
                for f in range(_F):
                    e00 = plsc.load_gather(tblv, [c00 + f])
                    e10 = plsc.load_gather(tblv, [c10 + f])
                    e01 = plsc.load_gather(tblv, [c01 + f])
                    e11 = plsc.load_gather(tblv, [c11 + f])
                    va = e00 + (e10 - e00) * wx
                    vb = e01 + (e11 - e01) * wx
                    v = va + (vb - va) * wy
                    plsc.store_scatter(
                        ob, [iota32 + (g512 + jnp.int32(2 * l + f))], v)

        for l in range(_NSMALL, _NUM_LEVELS):
            bl = l - _NSMALL
            copies[bl].wait()

            @pl.loop(0, _GRP)
            def _lerp(g, l=l, bl=bl):
                g16 = g.astype(jnp.int32) * jnp.int32(16)
                g512 = g.astype(jnp.int32) * jnp.int32(512)
                wx = wb[bl, 0, pl.ds(g16, 16)]
                wy = wb[bl, 1, pl.ds(g16, 16)]
                iota32 = iota * 32
                for f in range(_F):
                    e00 = rows[bl, pl.ds(g16 + jnp.int32((0 + f) * 128), 16)]
                    e10 = rows[bl, pl.ds(g16 + jnp.int32((2 + f) * 128), 16)]
                    e01 = rows[bl, pl.ds(g16 + jnp.int32((4 + f) * 128), 16)]
                    e11 = rows[bl, pl.ds(g16 + jnp.int32((6 + f) * 128), 16)]
                    va = e00 + (e10 - e00) * wx
                    vb = e01 + (e11 - e01) * wx
                    v = va + (vb - va) * wy
                    plsc.store_scatter(
                        ob, [iota32 + (g512 + jnp.int32(2 * l + f))], v)

        pltpu.sync_copy(ob, out_hbm.at[pl.ds(base * 32, 32 * _CHUNK)])


def kernel(x, tables):
    batch = x.shape[0]
    assert batch % (_NW * _CHUNK) == 0
    from jax._src import config as _jcfg
    with _jcfg.enable_x64(False):
        return _build_and_run(x, tables, batch)


def _build_and_run(x, tables, batch):
    mesh = plsc.VectorSubcoreMesh(core_axis_name="c", subcore_axis_name="s")
    cp = pltpu.CompilerParams()
    if "needs_layout_passes" in pltpu.CompilerParams.__dataclass_fields__:
        cp = dataclasses.replace(cp, needs_layout_passes=False)
    if "use_tc_tiling_on_sc" in pltpu.CompilerParams.__dataclass_fields__:
        cp = dataclasses.replace(cp, use_tc_tiling_on_sc=False)
    run = pl.kernel(
        _body,
        out_type=jax.ShapeDtypeStruct((batch * _OUTD,), jnp.float32),
        mesh=mesh,
        scratch_types=[
            pltpu.VMEM((2 * _CHUNK,), jnp.float32),
            pltpu.VMEM((2, _CHUNK), jnp.float32),
            pltpu.VMEM((_TBLV_WORDS,), jnp.float32),
            pltpu.VMEM((_NBIG, 4 * _F * _CHUNK), jnp.int32),
            pltpu.VMEM((_NBIG, 2, _CHUNK), jnp.float32),
            pltpu.VMEM((_NBIG, 4 * _F * _CHUNK), jnp.float32),
            pltpu.VMEM((_OUTD * _CHUNK,), jnp.float32),
            pltpu.SemaphoreType.DMA,
        ],
        compiler_params=cp,
    )
    x_flat = x.reshape(-1)
    tbl_flat = tuple(t.reshape(-1) for t in tables)
    out_flat = run(x_flat, *tbl_flat)
    return out_flat.reshape(batch, _OUTD)


# speedup vs baseline: 66.2715x; 1.0145x over previous
import dataclasses
import functools

import numpy as np
import jax
import jax.numpy as jnp
from jax import lax
from jax.experimental import pallas as pl
from jax.experimental.pallas import tpu as pltpu
from jax.experimental.pallas import tpu_sc as plsc

_NUM_LEVELS = 16
_BASE_RES = 16
_MAX_RES = 2048
_HASHMAP_SIZE = 2 ** 19
_gb = np.exp((np.log(_MAX_RES) - np.log(_BASE_RES)) / (_NUM_LEVELS - 1))
_RESOLUTIONS = [int(np.floor(_BASE_RES * _gb ** i)) for i in range(_NUM_LEVELS)]
_TABLE_SIZES = [min(r * r, _HASHMAP_SIZE) for r in _RESOLUTIONS]
_P1 = 2654435761
_P2 = 805459861
_P1H, _P1L = _P1 >> 16, _P1 & 0xFFFF
_P2H, _P2L = _P2 >> 16, _P2 & 0xFFFF

_NC, _NS = 2, 16
_NW = _NC * _NS
_CHUNK = 256
_GRP = _CHUNK // 16
_F = 2
_OUTD = _NUM_LEVELS * _F
_NSMALL = 7
_NBIG = _NUM_LEVELS - _NSMALL
_TBL_OFF = []
_acc = 0
for _t in _TABLE_SIZES[:_NSMALL]:
    _TBL_OFF.append(_acc)
    _acc += 2 * _t
_TBLV_WORDS = _acc


def _corner_pieces(c, ph, pl_):
    t = c * ph
    u = c * pl_
    lo = (t << 16) + u
    hi = (t + (u >> 16)) >> 16
    return hi, lo


def _combine(hi_a, lo_a, hi_b, lo_b, tsize):
    lo = lo_a ^ lo_b
    if tsize & (tsize - 1) == 0:
        return lo & (tsize - 1)
    hi = hi_a ^ hi_b
    m = lo & 0xFFFF
    nv = (lo >> 16) & 0xFFFF
    n1 = nv >> 8
    n0 = nv & 0xFF
    r32 = (1 << 32) % tsize
    r24 = (1 << 24) % tsize
    r16 = (1 << 16) % tsize
    s = hi * r32 + n1 * r24 + n0 * r16 + m
    inv_t = float(np.float32(1.0) / np.float32(tsize))
    q = (s.astype(jnp.float32) * inv_t).astype(jnp.int32)
    r = s - q * tsize
    r = jnp.where(r < 0, r + tsize, r)
    r = jnp.where(r >= tsize, r - tsize, r)
    return r


def _body(x_hbm, *refs):
    tables = refs[:_NUM_LEVELS]
    out_hbm = refs[_NUM_LEVELS]
    xb, normb, tblv, idxb, wb, rows, ob, sem = refs[_NUM_LEVELS + 1:]

    batch = x_hbm.shape[0] // 2
    rows_per_w = batch // _NW
    nchunks = rows_per_w // _CHUNK

    wid = (lax.axis_index("s").astype(jnp.int32) * jnp.int32(_NC)
           + lax.axis_index("c").astype(jnp.int32))
    iota = lax.iota(jnp.int32, 16)
    iota2 = iota * 2

    for l in range(_NSMALL):
        pltpu.sync_copy(tables[l], tblv.at[pl.ds(_TBL_OFF[l], 2 * _TABLE_SIZES[l])])

    @pl.loop(0, nchunks)
    def _chunk(ci):
        base = wid * jnp.int32(rows_per_w) + ci.astype(jnp.int32) * jnp.int32(_CHUNK)
        pltpu.sync_copy(x_hbm.at[pl.ds(base * 2, 2 * _CHUNK)], xb)

        @pl.loop(0, _GRP)
        def _norm(g):
            g16 = g.astype(jnp.int32) * jnp.int32(16)
            g32 = g.astype(jnp.int32) * jnp.int32(32)
            xs = plsc.load_gather(xb, [iota2 + g32])
            ys = plsc.load_gather(xb, [iota2 + (g32 + jnp.int32(1))])
            nx = jnp.minimum(jnp.maximum(xs * 0.5 + 0.5, 0.0), 1.0)
            ny = jnp.minimum(jnp.maximum(ys * 0.5 + 0.5, 0.0), 1.0)
            normb[0, pl.ds(g16, 16)] = nx
            normb[1, pl.ds(g16, 16)] = ny

        copies = []
        for l in range(_NSMALL, _NUM_LEVELS):
            res = _RESOLUTIONS[l]
            tsize = _TABLE_SIZES[l]
            bl = l - _NSMALL

            @pl.loop(0, _GRP)
            def _idx(g, bl=bl, res=res, tsize=tsize):
                g16 = g.astype(jnp.int32) * jnp.int32(16)
                nx = normb[0, pl.ds(g16, 16)]
                ny = normb[1, pl.ds(g16, 16)]
                sx = nx * float(res)
                sy = ny * float(res)
                x0 = sx.astype(jnp.int32)
                y0 = sy.astype(jnp.int32)
                wx = sx - x0.astype(jnp.float32)
                wy = sy - y0.astype(jnp.float32)
                x1 = jnp.minimum(x0 + 1, res - 1)
                y1 = jnp.minimum(y0 + 1, res - 1)
                x0c = jnp.minimum(x0, res - 1)
                y0c = jnp.minimum(y0, res - 1)
                ha0, la0 = _corner_pieces(x0c, _P1H, _P1L)
                ha1, la1 = _corner_pieces(x1, _P1H, _P1L)
                hb0, lb0 = _corner_pieces(y0c, _P2H, _P2L)
                hb1, lb1 = _corner_pieces(y1, _P2H, _P2L)
                c00 = _combine(ha0, la0, hb0, lb0, tsize) * 2
                c10 = _combine(ha1, la1, hb0, lb0, tsize) * 2
                c01 = _combine(ha0, la0, hb1, lb1, tsize) * 2
                c11 = _combine(ha1, la1, hb1, lb1, tsize) * 2
                idxb[bl, pl.ds(g16, 16)] = c00
                idxb[bl, pl.ds(g16 + jnp.int32(1 * _CHUNK), 16)] = c00 + 1
                idxb[bl, pl.ds(g16 + jnp.int32(2 * _CHUNK), 16)] = c10
                idxb[bl, pl.ds(g16 + jnp.int32(3 * _CHUNK), 16)] = c10 + 1
                idxb[bl, pl.ds(g16 + jnp.int32(4 * _CHUNK), 16)] = c01
                idxb[bl, pl.ds(g16 + jnp.int32(5 * _CHUNK), 16)] = c01 + 1
                idxb[bl, pl.ds(g16 + jnp.int32(6 * _CHUNK), 16)] = c11
                idxb[bl, pl.ds(g16 + jnp.int32(7 * _CHUNK), 16)] = c11 + 1
                wb[bl, 0, pl.ds(g16, 16)] = wx
                wb[bl, 1, pl.ds(g16, 16)] = wy

            copies.append(pltpu.async_copy(
                tables[l].at[idxb.at[bl]], rows.at[bl], sem))

        for l in range(_NSMALL):
            res = _RESOLUTIONS[l]
            tsize = _TABLE_SIZES[l]
            off = _TBL_OFF[l]

            @pl.loop(0, _GRP)
            def _small(g, l=l, res=res, tsize=tsize, off=off):
                g16 = g.astype(jnp.int32) * jnp.int32(16)
                nx = normb[0, pl.ds(g16, 16)]
                ny = normb[1, pl.ds(g16, 16)]
                sx = nx * float(res)
                sy = ny * float(res)
                x0 = sx.astype(jnp.int32)
                y0 = sy.astype(jnp.int32)
                wx = sx - x0.astype(jnp.float32)
                wy = sy - y0.astype(jnp.float32)
                x1 = jnp.minimum(x0 + 1, res - 1)
                y1 = jnp.minimum(y0 + 1, res - 1)
                x0c = jnp.minimum(x0, res - 1)
                y0c = jnp.minimum(y0, res - 1)
                ha0, la0 = _corner_pieces(x0c, _P1H, _P1L)
                ha1, la1 = _corner_pieces(x1, _P1H, _P1L)
                hb0, lb0 = _corner_pieces(y0c, _P2H, _P2L)
                hb1, lb1 = _corner_pieces(y1, _P2H, _P2L)
                c00 = _combine(ha0, la0, hb0, lb0, tsize) * 2 + off
                c10 = _combine(ha1, la1, hb0, lb0, tsize) * 2 + off
                c01 = _combine(ha0, la0, hb1, lb1, tsize) * 2 + off
                c11 = _combine(ha1, la1, hb1, lb1, tsize) * 2 + off
                g512 = g16 * jnp.int32(32)
                iota32 = iota * 32
                for f in range(_F):
                    e00 = plsc.load_gather(tblv, [c00 + f])
                    e10 = plsc.load_gather(tblv, [c10 + f])
                    e01 = plsc.load_gather(tblv, [c01 + f])
                    e11 = plsc.load_gather(tblv, [c11 + f])
                    va = e00 + (e10 - e00) * wx
                    vb = e01 + (e11 - e01) * wx
                    v = va + (vb - va) * wy
                    plsc.store_scatter(
                        ob, [iota32 + (g512 + jnp.int32(2 * l + f))], v)

        for l in range(_NSMALL, _NUM_LEVELS):
            bl = l - _NSMALL
            copies[bl].wait()

            @pl.loop(0, _GRP)
            def _lerp(g, l=l, bl=bl):
                g16 = g.astype(jnp.int32) * jnp.int32(16)
                wx = wb[bl, 0, pl.ds(g16, 16)]
                wy = wb[bl, 1, pl.ds(g16, 16)]
                g512 = g16 * jnp.int32(32)
                iota32 = iota * 32
                for f in range(_F):
                    e00 = rows[bl, pl.ds(g16 + jnp.int32((0 + f) * _CHUNK), 16)]
                    e10 = rows[bl, pl.ds(g16 + jnp.int32((2 + f) * _CHUNK), 16)]
                    e01 = rows[bl, pl.ds(g16 + jnp.int32((4 + f) * _CHUNK), 16)]
                    e11 = rows[bl, pl.ds(g16 + jnp.int32((6 + f) * _CHUNK), 16)]
                    va = e00 + (e10 - e00) * wx
                    vb = e01 + (e11 - e01) * wx
                    v = va + (vb - va) * wy
                    plsc.store_scatter(
                        ob, [iota32 + (g512 + jnp.int32(2 * l + f))], v)

        pltpu.sync_copy(ob, out_hbm.at[pl.ds(base * 32, 32 * _CHUNK)])


def kernel(x, tables):
    batch = x.shape[0]
    assert batch % (_NW * _CHUNK) == 0
    from jax._src import config as _jcfg
    with _jcfg.enable_x64(False):
        return _build_and_run(x, tables, batch)


def _build_and_run(x, tables, batch):
    mesh = plsc.VectorSubcoreMesh(core_axis_name="c", subcore_axis_name="s")
    cp = pltpu.CompilerParams()
    if "needs_layout_passes" in pltpu.CompilerParams.__dataclass_fields__:
        cp = dataclasses.replace(cp, needs_layout_passes=False)
    if "use_tc_tiling_on_sc" in pltpu.CompilerParams.__dataclass_fields__:
        cp = dataclasses.replace(cp, use_tc_tiling_on_sc=False)
    run = pl.kernel(
        _body,
        out_type=jax.ShapeDtypeStruct((batch * _OUTD,), jnp.float32),
        mesh=mesh,
        scratch_types=[
            pltpu.VMEM((2 * _CHUNK,), jnp.float32),
            pltpu.VMEM((2, _CHUNK), jnp.float32),
            pltpu.VMEM((_TBLV_WORDS,), jnp.float32),
            pltpu.VMEM((_NBIG, 4 * _F * _CHUNK), jnp.int32),
            pltpu.VMEM((_NBIG, 2, _CHUNK), jnp.float32),
            pltpu.VMEM((_NBIG, 4 * _F * _CHUNK), jnp.float32),
            pltpu.VMEM((_OUTD * _CHUNK,), jnp.float32),
            pltpu.SemaphoreType.DMA,
        ],
        compiler_params=cp,
    )
    x_flat = x.reshape(-1)
    tbl_flat = tuple(t.reshape(-1) for t in tables)
    out_flat = run(x_flat, *tbl_flat)
    return out_flat.reshape(batch, _OUTD)
